# baseline (speedup 1.0000x reference)
import functools

import jax
import jax.numpy as jnp
from jax import lax
from jax.experimental import pallas as pl
from jax.experimental.pallas import tpu as pltpu
from jax.experimental.pallas import tpu_sc as plsc

EMB = 32
LANES = 16
NUM_CORES = 2
NUM_SUBCORES = 16
IDX_CHUNK = 128


def _build_fm_kernel(batch):
    num_workers = NUM_CORES * NUM_SUBCORES
    bpw = batch // num_workers
    n_chunks = bpw // IDX_CHUNK
    n_groups = bpw // LANES
    mesh = plsc.VectorSubcoreMesh(core_axis_name="c", subcore_axis_name="s")

    @functools.partial(
        pl.kernel,
        mesh=mesh,
        compiler_params=pltpu.CompilerParams(
            needs_layout_passes=False, use_tc_tiling_on_sc=False),
        out_type=jax.ShapeDtypeStruct((batch,), jnp.float32),
        scratch_types=[
            pltpu.VMEM((n_chunks, IDX_CHUNK), jnp.int32),
            pltpu.VMEM((n_chunks, IDX_CHUNK), jnp.int32),
            pltpu.VMEM((bpw, EMB), jnp.float32),
            pltpu.VMEM((bpw, EMB), jnp.float32),
            pltpu.VMEM((2 * EMB * LANES,), jnp.float32),
            pltpu.VMEM((2 * EMB * LANES,), jnp.float32),
            pltpu.VMEM((2 * EMB * LANES,), jnp.float32),
            pltpu.VMEM((LANES,), jnp.float32),
            pltpu.VMEM((bpw,), jnp.float32),
            pltpu.SemaphoreType.DMA,
        ],
    )
    def fm_kernel(uids_hbm, iids_hbm, utab_hbm, itab_hbm,
                  wl_hbm, wf_hbm, wfh_hbm, b_hbm, out_hbm,
                  uidx_v, iidx_v, urows_v, irows_v,
                  wl_v, wf_v, wfh_v, b_v, out_v, sem):
        wid = lax.axis_index("s") * NUM_CORES + lax.axis_index("c")
        base = wid * bpw

        pltpu.sync_copy(uids_hbm.at[wid], uidx_v)
        pltpu.sync_copy(iids_hbm.at[wid], iidx_v)
        pltpu.sync_copy(wl_hbm, wl_v)
        pltpu.sync_copy(wf_hbm, wf_v)
        pltpu.sync_copy(wfh_hbm, wfh_v)
        pltpu.sync_copy(b_hbm, b_v)

        copies = []
        for j in range(n_chunks):
            rows = pl.ds(j * IDX_CHUNK, IDX_CHUNK)
            copies.append(pltpu.async_copy(
                utab_hbm.at[uidx_v.at[j]], urows_v.at[rows], sem))
            copies.append(pltpu.async_copy(
                itab_hbm.at[iidx_v.at[j]], irows_v.at[rows], sem))
        for c in copies:
            c.wait()

        lane_iota = lax.iota(jnp.int32, LANES)
        bias = b_v[...]

        def group_body(g, carry):
            ridx = g * LANES + lane_iota
            acc_lq = bias
            acc_s = jnp.zeros((LANES,), jnp.float32)
            for d in range(EMB):
                dvec = jnp.full((LANES,), d, jnp.int32)
                uc = plsc.load_gather(urows_v, [ridx, dvec])
                ic = plsc.load_gather(irows_v, [ridx, dvec])
                du = d * LANES
                di = (EMB + d) * LANES
                wl_u = wl_v[pl.ds(du, LANES)]
                wf_u = wf_v[pl.ds(du, LANES)]
                wh_u = wfh_v[pl.ds(du, LANES)]
                wl_i = wl_v[pl.ds(di, LANES)]
                wf_i = wf_v[pl.ds(di, LANES)]
                wh_i = wfh_v[pl.ds(di, LANES)]
                acc_lq = acc_lq + uc * (wl_u + uc * wh_u)
                acc_s = acc_s + uc * wf_u
                acc_lq = acc_lq + ic * (wl_i + ic * wh_i)
                acc_s = acc_s + ic * wf_i
            out_v[pl.ds(g * LANES, LANES)] = acc_lq + (acc_s * acc_s) * 0.5
            return carry

        lax.fori_loop(0, n_groups, group_body, 0)
        pltpu.sync_copy(out_v, out_hbm.at[pl.ds(base, bpw)])

    return fm_kernel


def kernel(user_ids, item_ids, user_table, item_table, W_lin, b_lin, W_fm):
    batch = user_ids.shape[0]
    num_workers = NUM_CORES * NUM_SUBCORES
    uids = user_ids.astype(jnp.int32).reshape(num_workers, -1, IDX_CHUNK)
    iids = item_ids.astype(jnp.int32).reshape(num_workers, -1, IDX_CHUNK)
    wl = jnp.repeat(W_lin.reshape(-1), LANES)
    wf = jnp.repeat(W_fm.reshape(-1), LANES)
    wfh = -0.5 * wf
    bias = jnp.broadcast_to(b_lin.reshape(1), (LANES,))
    return _build_fm_kernel(batch)(
        uids, iids, user_table, item_table, wl, wf, wfh, bias)
